# Initial kernel scaffold; baseline (speedup 1.0000x reference)
import jax
import jax.numpy as jnp
from jax.experimental import pallas as pl


def kernel(x, edge_index, W1, b1, W2, b2, W3, b3):
    raise NotImplementedError("write your pallas kernel here")


# speedup vs baseline: 13.2831x; 13.2831x over previous
import functools

import jax
import jax.numpy as jnp
from jax import lax
from jax.experimental import pallas as pl
from jax.experimental.pallas import tpu as pltpu
from jax.experimental.pallas import tpu_sc as plsc

N = 10000
E = 160000
K = 128
CH = 80
CH2 = CH // 2
EPAD = 16 * CH * K
NACC = 10240
SLICE = NACC // 16
CO = 624
REM = N - 16 * CO

_mesh = plsc.VectorSubcoreMesh(core_axis_name="c", subcore_axis_name="s")


@functools.partial(
    pl.kernel,
    out_type=jax.ShapeDtypeStruct((NACC,), jnp.float32),
    mesh=_mesh,
    scratch_types=[
        pltpu.VMEM((CH, K), jnp.int32),
        pltpu.VMEM((K,), jnp.float32),
        pltpu.VMEM((SLICE,), jnp.float32),
        pltpu.VMEM_SHARED((NACC,), jnp.float32),
        pltpu.SemaphoreType.DMA,
    ],
)
def _deg_kernel(dst_hbm, out_hbm, dst_buf, ones_buf, res_buf, acc_sh, sem):
    c = lax.axis_index("c")
    s = lax.axis_index("s")
    for i in range(K // 16):
        ones_buf[pl.ds(i * 16, 16)] = jnp.full((16,), 1.0, jnp.float32)
    for i in range(SLICE // 16):
        res_buf[pl.ds(i * 16, 16)] = jnp.zeros((16,), jnp.float32)
    pltpu.sync_copy(res_buf, acc_sh.at[pl.ds(s * SLICE, SLICE)])
    plsc.subcore_barrier()
    pltpu.sync_copy(dst_hbm.at[s], dst_buf)

    def body(j, carry):
        pltpu.sync_copy(ones_buf, acc_sh.at[dst_buf.at[j]], add=True)
        return carry

    lax.fori_loop(0, CH, body, 0)
    plsc.subcore_barrier()
    pltpu.sync_copy(acc_sh.at[pl.ds(s * SLICE, SLICE)], res_buf)
    for i in range(SLICE // 16):
        res_buf[pl.ds(i * 16, 16)] = res_buf[pl.ds(i * 16, 16)] + 1.0

    @pl.when(c == 0)
    def _():
        pltpu.sync_copy(res_buf, out_hbm.at[pl.ds(s * SLICE, SLICE)])


@functools.partial(
    pl.kernel,
    out_type=jax.ShapeDtypeStruct((2 * N, 128), jnp.float32),
    mesh=_mesh,
    scratch_types=[
        pltpu.VMEM((CH, K), jnp.int32),
        pltpu.VMEM((CH, K), jnp.int32),
        pltpu.VMEM((K, 128), jnp.float32),
        pltpu.VMEM_SHARED((NACC, 128), jnp.float32),
        pltpu.SemaphoreType.DMA,
    ],
)
def _scatter_fsplit(g_hbm, src_hbm, dst_hbm, out_hbm,
                    src_buf, dst_buf, rows, acc_sh, sem):
    c = lax.axis_index("c")
    s = lax.axis_index("s")
    base = c * N
    pltpu.sync_copy(g_hbm.at[pl.ds(base + s * CO, CO)],
                    acc_sh.at[pl.ds(s * CO, CO)])

    @pl.when(s == 0)
    def _():
        pltpu.sync_copy(g_hbm.at[pl.ds(base + 16 * CO, REM)],
                        acc_sh.at[pl.ds(16 * CO, REM)])

    pltpu.sync_copy(src_hbm.at[c, s], src_buf)
    pltpu.sync_copy(dst_hbm.at[s], dst_buf)
    plsc.subcore_barrier()

    def body(j, carry):
        pltpu.async_copy(g_hbm.at[src_buf.at[j]], rows, sem).wait()
        pltpu.sync_copy(rows, acc_sh.at[dst_buf.at[j]], add=True)
        return carry

    lax.fori_loop(0, CH, body, 0)
    plsc.subcore_barrier()
    pltpu.sync_copy(acc_sh.at[pl.ds(s * CO, CO)],
                    out_hbm.at[pl.ds(base + s * CO, CO)])

    @pl.when(s == 0)
    def _():
        pltpu.sync_copy(acc_sh.at[pl.ds(16 * CO, REM)],
                        out_hbm.at[pl.ds(base + 16 * CO, REM)])


@functools.partial(
    pl.kernel,
    out_type=jax.ShapeDtypeStruct((2 * N, 128), jnp.float32),
    mesh=_mesh,
    scratch_types=[
        pltpu.VMEM((CH2, K), jnp.int32),
        pltpu.VMEM((CH2, K), jnp.int32),
        pltpu.VMEM((K, 128), jnp.float32),
        pltpu.VMEM_SHARED((NACC, 128), jnp.float32),
        pltpu.SemaphoreType.DMA,
    ],
)
def _scatter_esplit(g_hbm, src_hbm, dst_hbm, out_hbm,
                    src_buf, dst_buf, rows, acc_sh, sem):
    c = lax.axis_index("c")
    s = lax.axis_index("s")
    base = c * N
    for r in range(K):
        for q in range(8):
            rows[r, pl.ds(q * 16, 16)] = jnp.zeros((16,), jnp.float32)
    for t in range(SLICE // K):
        pltpu.sync_copy(rows, acc_sh.at[pl.ds(s * SLICE + t * K, K)])
    pltpu.sync_copy(src_hbm.at[c, s], src_buf)
    pltpu.sync_copy(dst_hbm.at[c, s], dst_buf)
    plsc.subcore_barrier()

    def body(j, carry):
        pltpu.async_copy(g_hbm.at[src_buf.at[j]], rows, sem).wait()
        pltpu.sync_copy(rows, acc_sh.at[dst_buf.at[j]], add=True)
        return carry

    lax.fori_loop(0, CH2, body, 0)
    plsc.subcore_barrier()
    pltpu.sync_copy(acc_sh.at[pl.ds(s * CO, CO)],
                    out_hbm.at[pl.ds(base + s * CO, CO)])

    @pl.when(s == 0)
    def _():
        pltpu.sync_copy(acc_sh.at[pl.ds(16 * CO, REM)],
                        out_hbm.at[pl.ds(base + 16 * CO, REM)])


_RB = 1000


def _tc_first(x, deg, w1):
    f_in, f_out = w1.shape
    dh = f_out // 2

    def body(x_ref, deg_ref, w_ref, g_ref, dinv_ref):
        dv = lax.rsqrt(deg_ref[...])
        m = jnp.dot(x_ref[...], w_ref[...], preferred_element_type=jnp.float32)
        g_ref[...] = (m * dv)[None]
        dinv_ref[...] = dv

    return pl.pallas_call(
        body,
        grid=(N // _RB, 2),
        in_specs=[
            pl.BlockSpec((_RB, f_in), lambda r, c: (r, 0)),
            pl.BlockSpec((_RB, 1), lambda r, c: (r, 0)),
            pl.BlockSpec((f_in, dh), lambda r, c: (0, c)),
        ],
        out_specs=[
            pl.BlockSpec((1, _RB, dh), lambda r, c: (c, r, 0)),
            pl.BlockSpec((_RB, 1), lambda r, c: (r, 0)),
        ],
        out_shape=[
            jax.ShapeDtypeStruct((2, N, dh), jnp.float32),
            jax.ShapeDtypeStruct((N, 1), jnp.float32),
        ],
    )(x, deg, w1)


def _tc_second(sa, sb, dinv, b, w):
    f_in, f_out = w.shape
    dh_in = sa.shape[1]

    def body(sa_ref, sb_ref, dinv_ref, b_ref, w_ref, g_ref):
        dv = dinv_ref[...]
        h = jnp.concatenate([sa_ref[...], sb_ref[...]], axis=1) * dv + b_ref[...]
        h = jnp.maximum(h, 0.0)
        g_ref[...] = jnp.dot(h, w_ref[...],
                             preferred_element_type=jnp.float32) * dv

    return pl.pallas_call(
        body,
        grid=(N // _RB,),
        in_specs=[
            pl.BlockSpec((_RB, dh_in), lambda r: (r, 0)),
            pl.BlockSpec((_RB, dh_in), lambda r: (r, 0)),
            pl.BlockSpec((_RB, 1), lambda r: (r, 0)),
            pl.BlockSpec((1, f_in), lambda r: (0, 0)),
            pl.BlockSpec((f_in, f_out), lambda r: (0, 0)),
        ],
        out_specs=pl.BlockSpec((_RB, f_out), lambda r: (r, 0)),
        out_shape=jax.ShapeDtypeStruct((N, f_out), jnp.float32),
    )(sa, sb, dinv, b, w)


def _tc_third(pa, pb, g, dinv, b):
    f = g.shape[1]

    def body(pa_ref, pb_ref, g_ref, dinv_ref, b_ref, o_ref):
        dv = dinv_ref[...]
        s = pa_ref[...] + pb_ref[...] + g_ref[...]
        h = jnp.maximum(s * dv + b_ref[...], 0.0)
        o_ref[...] = h * dv

    return pl.pallas_call(
        body,
        grid=(N // _RB,),
        in_specs=[
            pl.BlockSpec((_RB, f), lambda r: (r, 0)),
            pl.BlockSpec((_RB, f), lambda r: (r, 0)),
            pl.BlockSpec((_RB, f), lambda r: (r, 0)),
            pl.BlockSpec((_RB, 1), lambda r: (r, 0)),
            pl.BlockSpec((1, f), lambda r: (0, 0)),
        ],
        out_specs=pl.BlockSpec((_RB, f), lambda r: (r, 0)),
        out_shape=jax.ShapeDtypeStruct((N, f), jnp.float32),
    )(pa, pb, g, dinv, b)


def _tc_last(pa, pb, g, dinv, w, b):
    f_in, f_out = w.shape

    def body(pa_ref, pb_ref, g_ref, dinv_ref, w_ref, b_ref, o_ref):
        s = pa_ref[...] + pb_ref[...] + g_ref[...]
        m = jnp.dot(s, w_ref[...], preferred_element_type=jnp.float32)
        o_ref[...] = m * dinv_ref[...] + b_ref[...]

    return pl.pallas_call(
        body,
        grid=(N // _RB,),
        in_specs=[
            pl.BlockSpec((_RB, f_in), lambda r: (r, 0)),
            pl.BlockSpec((_RB, f_in), lambda r: (r, 0)),
            pl.BlockSpec((_RB, f_in), lambda r: (r, 0)),
            pl.BlockSpec((_RB, 1), lambda r: (r, 0)),
            pl.BlockSpec((f_in, f_out), lambda r: (0, 0)),
            pl.BlockSpec((1, f_out), lambda r: (0, 0)),
        ],
        out_specs=pl.BlockSpec((_RB, f_out), lambda r: (r, 0)),
        out_shape=jax.ShapeDtypeStruct((N, f_out), jnp.float32),
    )(pa, pb, g, dinv, w, b)


def kernel(x, edge_index, W1, b1, W2, b2, W3, b3):
    src = edge_index[0].astype(jnp.int32)
    dst = edge_index[1].astype(jnp.int32)
    pad = EPAD - E
    pad_src = (jnp.arange(pad, dtype=jnp.int32) * 37) % N
    pad_dst = N + (jnp.arange(pad, dtype=jnp.int32) % (NACC - N))
    src_p = jnp.concatenate([src, pad_src])
    dst_p = jnp.concatenate([dst, pad_dst])
    src2 = jnp.stack([src_p, src_p + N]).reshape(2, 16, CH, K)
    dst3 = dst_p.reshape(16, CH, K)
    src_h = src_p.reshape(2, 16, CH2, K)
    dst_h = dst_p.reshape(2, 16, CH2, K)

    deg = _deg_kernel(dst3)[:N].reshape(N, 1)

    g1, dinv = _tc_first(x, deg, W1)
    s1 = _scatter_fsplit(g1.reshape(2 * N, 128), src2, dst3).reshape(2, N, 128)
    g2 = _tc_second(s1[0], s1[1], dinv, b1.reshape(1, -1), W2)
    p2 = _scatter_esplit(g2, src_h, dst_h).reshape(2, N, 128)
    g3 = _tc_third(p2[0], p2[1], g2, dinv, b2.reshape(1, -1))
    p3 = _scatter_esplit(g3, src_h, dst_h).reshape(2, N, 128)
    return _tc_last(p3[0], p3[1], g3, dinv, W3, b3.reshape(1, -1))


# speedup vs baseline: 18.3792x; 1.3836x over previous
import functools

import jax
import jax.numpy as jnp
from jax import lax
from jax.experimental import pallas as pl
from jax.experimental.pallas import tpu as pltpu
from jax.experimental.pallas import tpu_sc as plsc

N = 10000
E = 160000
K = 112
CH = 90
CH2 = CH // 2
EPAD = 16 * CH * K
NACC = 10240
SLICE = NACC // 16
CO = 624
REM = N - 16 * CO

_mesh = plsc.VectorSubcoreMesh(core_axis_name="c", subcore_axis_name="s")


@functools.partial(
    pl.kernel,
    out_type=jax.ShapeDtypeStruct((NACC,), jnp.float32),
    mesh=_mesh,
    scratch_types=[
        pltpu.VMEM((CH, K), jnp.int32),
        pltpu.VMEM((K,), jnp.float32),
        pltpu.VMEM((SLICE,), jnp.float32),
        pltpu.VMEM_SHARED((NACC,), jnp.float32),
        pltpu.SemaphoreType.DMA,
    ],
)
def _deg_kernel(dst_hbm, out_hbm, dst_buf, ones_buf, res_buf, acc_sh, sem):
    c = lax.axis_index("c")
    s = lax.axis_index("s")
    for i in range(K // 16):
        ones_buf[pl.ds(i * 16, 16)] = jnp.full((16,), 1.0, jnp.float32)
    for i in range(SLICE // 16):
        res_buf[pl.ds(i * 16, 16)] = jnp.zeros((16,), jnp.float32)
    pltpu.sync_copy(res_buf, acc_sh.at[pl.ds(s * SLICE, SLICE)])
    plsc.subcore_barrier()
    pltpu.sync_copy(dst_hbm.at[s], dst_buf)

    def body(j, carry):
        pltpu.sync_copy(ones_buf, acc_sh.at[dst_buf.at[j]], add=True)
        return carry

    lax.fori_loop(0, CH, body, 0)
    plsc.subcore_barrier()
    pltpu.sync_copy(acc_sh.at[pl.ds(s * SLICE, SLICE)], res_buf)
    for i in range(SLICE // 16):
        res_buf[pl.ds(i * 16, 16)] = res_buf[pl.ds(i * 16, 16)] + 1.0

    @pl.when(c == 0)
    def _():
        pltpu.sync_copy(res_buf, out_hbm.at[pl.ds(s * SLICE, SLICE)])


def _edge_loop(g_hbm, src_row, dst_row, acc_sh, sidx, didx, rows,
               msi, mdi, mr, n):
    def idx_issue(j, b):
        pltpu.async_copy(src_row(j), sidx[b], msi[b])
        pltpu.async_copy(dst_row(j), didx[b], mdi[b])

    def idx_wait(b):
        pltpu.make_async_copy(src_row(0), sidx[b], msi[b]).wait()
        pltpu.make_async_copy(dst_row(0), didx[b], mdi[b]).wait()

    idx_issue(0, 0)
    idx_issue(1, 1)
    idx_wait(0)
    pltpu.async_copy(g_hbm.at[sidx[0]], rows[0], mr[0])

    def body(t, carry):
        j0 = t * 3
        for b in range(3):
            j = j0 + b
            b1 = (b + 1) % 3
            b2 = (b + 2) % 3

            @pl.when(j + 2 < n)
            def _():
                idx_issue(j + 2, b2)

            @pl.when(j + 1 < n)
            def _():
                idx_wait(b1)
                pltpu.async_copy(g_hbm.at[sidx[b1]], rows[b1], mr[b1])

            pltpu.make_async_copy(g_hbm.at[sidx[b]], rows[b], mr[b]).wait()
            pltpu.sync_copy(rows[b], acc_sh.at[didx[b]], add=True)
        return carry

    lax.fori_loop(0, n // 3, body, 0)


_SCAT_SCRATCH = [
    pltpu.VMEM((K,), jnp.int32),
    pltpu.VMEM((K,), jnp.int32),
    pltpu.VMEM((K,), jnp.int32),
    pltpu.VMEM((K,), jnp.int32),
    pltpu.VMEM((K,), jnp.int32),
    pltpu.VMEM((K,), jnp.int32),
    pltpu.VMEM((K, 128), jnp.float32),
    pltpu.VMEM((K, 128), jnp.float32),
    pltpu.VMEM((K, 128), jnp.float32),
    pltpu.VMEM_SHARED((NACC, 128), jnp.float32),
] + [pltpu.SemaphoreType.DMA] * 9


@functools.partial(
    pl.kernel,
    out_type=jax.ShapeDtypeStruct((2 * N, 128), jnp.float32),
    mesh=_mesh,
    scratch_types=_SCAT_SCRATCH,
)
def _scatter_fsplit(g_hbm, src_hbm, dst_hbm, out_hbm,
                    s0, s1, s2, d0, d1, d2, r0, r1, r2, acc_sh,
                    a0, a1, a2, b0, b1, b2, c0, c1, c2):
    c = lax.axis_index("c")
    s = lax.axis_index("s")
    base = c * N
    pltpu.sync_copy(g_hbm.at[pl.ds(base + s * CO, CO)],
                    acc_sh.at[pl.ds(s * CO, CO)])

    @pl.when(s == 0)
    def _():
        pltpu.sync_copy(g_hbm.at[pl.ds(base + 16 * CO, REM)],
                        acc_sh.at[pl.ds(16 * CO, REM)])

    plsc.subcore_barrier()
    _edge_loop(g_hbm,
               lambda j: src_hbm.at[c, s, j],
               lambda j: dst_hbm.at[s, j],
               acc_sh, (s0, s1, s2), (d0, d1, d2), (r0, r1, r2),
               (a0, a1, a2), (b0, b1, b2), (c0, c1, c2), CH)
    plsc.subcore_barrier()
    pltpu.sync_copy(acc_sh.at[pl.ds(s * CO, CO)],
                    out_hbm.at[pl.ds(base + s * CO, CO)])

    @pl.when(s == 0)
    def _():
        pltpu.sync_copy(acc_sh.at[pl.ds(16 * CO, REM)],
                        out_hbm.at[pl.ds(base + 16 * CO, REM)])


@functools.partial(
    pl.kernel,
    out_type=jax.ShapeDtypeStruct((2 * N, 128), jnp.float32),
    mesh=_mesh,
    scratch_types=_SCAT_SCRATCH,
)
def _scatter_esplit(g_hbm, src_hbm, dst_hbm, out_hbm,
                    s0, s1, s2, d0, d1, d2, r0, r1, r2, acc_sh,
                    a0, a1, a2, b0, b1, b2, c0, c1, c2):
    c = lax.axis_index("c")
    s = lax.axis_index("s")
    base = c * N
    for r in range(K):
        for q in range(8):
            r0[r, pl.ds(q * 16, 16)] = jnp.zeros((16,), jnp.float32)
    for t in range(SLICE // K):
        pltpu.sync_copy(r0, acc_sh.at[pl.ds(s * SLICE + t * K, K)])
    zrem = SLICE - (SLICE // K) * K
    if zrem:
        pltpu.sync_copy(r0.at[pl.ds(0, zrem)],
                        acc_sh.at[pl.ds(s * SLICE + (SLICE // K) * K, zrem)])
    plsc.subcore_barrier()
    _edge_loop(g_hbm,
               lambda j: src_hbm.at[c, s, j],
               lambda j: dst_hbm.at[c, s, j],
               acc_sh, (s0, s1, s2), (d0, d1, d2), (r0, r1, r2),
               (a0, a1, a2), (b0, b1, b2), (c0, c1, c2), CH2)
    plsc.subcore_barrier()
    pltpu.sync_copy(acc_sh.at[pl.ds(s * CO, CO)],
                    out_hbm.at[pl.ds(base + s * CO, CO)])

    @pl.when(s == 0)
    def _():
        pltpu.sync_copy(acc_sh.at[pl.ds(16 * CO, REM)],
                        out_hbm.at[pl.ds(base + 16 * CO, REM)])


_RB = 1000


def _tc_first(x, deg, w1):
    f_in, f_out = w1.shape
    dh = f_out // 2

    def body(x_ref, deg_ref, w_ref, g_ref, dinv_ref):
        dv = lax.rsqrt(deg_ref[...])
        m = jnp.dot(x_ref[...], w_ref[...], preferred_element_type=jnp.float32)
        g_ref[...] = (m * dv)[None]
        dinv_ref[...] = dv

    return pl.pallas_call(
        body,
        grid=(N // _RB, 2),
        in_specs=[
            pl.BlockSpec((_RB, f_in), lambda r, c: (r, 0)),
            pl.BlockSpec((_RB, 1), lambda r, c: (r, 0)),
            pl.BlockSpec((f_in, dh), lambda r, c: (0, c)),
        ],
        out_specs=[
            pl.BlockSpec((1, _RB, dh), lambda r, c: (c, r, 0)),
            pl.BlockSpec((_RB, 1), lambda r, c: (r, 0)),
        ],
        out_shape=[
            jax.ShapeDtypeStruct((2, N, dh), jnp.float32),
            jax.ShapeDtypeStruct((N, 1), jnp.float32),
        ],
    )(x, deg, w1)


def _tc_second(sa, sb, dinv, b, w):
    f_in, f_out = w.shape
    dh_in = sa.shape[1]

    def body(sa_ref, sb_ref, dinv_ref, b_ref, w_ref, g_ref):
        dv = dinv_ref[...]
        h = jnp.concatenate([sa_ref[...], sb_ref[...]], axis=1) * dv + b_ref[...]
        h = jnp.maximum(h, 0.0)
        g_ref[...] = jnp.dot(h, w_ref[...],
                             preferred_element_type=jnp.float32) * dv

    return pl.pallas_call(
        body,
        grid=(N // _RB,),
        in_specs=[
            pl.BlockSpec((_RB, dh_in), lambda r: (r, 0)),
            pl.BlockSpec((_RB, dh_in), lambda r: (r, 0)),
            pl.BlockSpec((_RB, 1), lambda r: (r, 0)),
            pl.BlockSpec((1, f_in), lambda r: (0, 0)),
            pl.BlockSpec((f_in, f_out), lambda r: (0, 0)),
        ],
        out_specs=pl.BlockSpec((_RB, f_out), lambda r: (r, 0)),
        out_shape=jax.ShapeDtypeStruct((N, f_out), jnp.float32),
    )(sa, sb, dinv, b, w)


def _tc_third(pa, pb, g, dinv, b):
    f = g.shape[1]

    def body(pa_ref, pb_ref, g_ref, dinv_ref, b_ref, o_ref):
        dv = dinv_ref[...]
        s = pa_ref[...] + pb_ref[...] + g_ref[...]
        h = jnp.maximum(s * dv + b_ref[...], 0.0)
        o_ref[...] = h * dv

    return pl.pallas_call(
        body,
        grid=(N // _RB,),
        in_specs=[
            pl.BlockSpec((_RB, f), lambda r: (r, 0)),
            pl.BlockSpec((_RB, f), lambda r: (r, 0)),
            pl.BlockSpec((_RB, f), lambda r: (r, 0)),
            pl.BlockSpec((_RB, 1), lambda r: (r, 0)),
            pl.BlockSpec((1, f), lambda r: (0, 0)),
        ],
        out_specs=pl.BlockSpec((_RB, f), lambda r: (r, 0)),
        out_shape=jax.ShapeDtypeStruct((N, f), jnp.float32),
    )(pa, pb, g, dinv, b)


def _tc_last(pa, pb, g, dinv, w, b):
    f_in, f_out = w.shape

    def body(pa_ref, pb_ref, g_ref, dinv_ref, w_ref, b_ref, o_ref):
        s = pa_ref[...] + pb_ref[...] + g_ref[...]
        m = jnp.dot(s, w_ref[...], preferred_element_type=jnp.float32)
        o_ref[...] = m * dinv_ref[...] + b_ref[...]

    return pl.pallas_call(
        body,
        grid=(N // _RB,),
        in_specs=[
            pl.BlockSpec((_RB, f_in), lambda r: (r, 0)),
            pl.BlockSpec((_RB, f_in), lambda r: (r, 0)),
            pl.BlockSpec((_RB, f_in), lambda r: (r, 0)),
            pl.BlockSpec((_RB, 1), lambda r: (r, 0)),
            pl.BlockSpec((f_in, f_out), lambda r: (0, 0)),
            pl.BlockSpec((1, f_out), lambda r: (0, 0)),
        ],
        out_specs=pl.BlockSpec((_RB, f_out), lambda r: (r, 0)),
        out_shape=jax.ShapeDtypeStruct((N, f_out), jnp.float32),
    )(pa, pb, g, dinv, w, b)


def kernel(x, edge_index, W1, b1, W2, b2, W3, b3):
    src = edge_index[0].astype(jnp.int32)
    dst = edge_index[1].astype(jnp.int32)
    pad = EPAD - E
    pad_src = (jnp.arange(pad, dtype=jnp.int32) * 37) % N
    pad_dst = N + (jnp.arange(pad, dtype=jnp.int32) % (NACC - N))
    src_p = jnp.concatenate([src, pad_src])
    dst_p = jnp.concatenate([dst, pad_dst])
    src2 = jnp.stack([src_p, src_p + N]).reshape(2, 16, CH, K)
    dst3 = dst_p.reshape(16, CH, K)
    src_h = src_p.reshape(2, 16, CH2, K)
    dst_h = dst_p.reshape(2, 16, CH2, K)

    deg = _deg_kernel(dst3)[:N].reshape(N, 1)

    g1, dinv = _tc_first(x, deg, W1)
    s1 = _scatter_fsplit(g1.reshape(2 * N, 128), src2, dst3).reshape(2, N, 128)
    g2 = _tc_second(s1[0], s1[1], dinv, b1.reshape(1, -1), W2)
    p2 = _scatter_esplit(g2, src_h, dst_h).reshape(2, N, 128)
    g3 = _tc_third(p2[0], p2[1], g2, dinv, b2.reshape(1, -1))
    p3 = _scatter_esplit(g3, src_h, dst_h).reshape(2, N, 128)
    return _tc_last(p3[0], p3[1], g3, dinv, W3, b3.reshape(1, -1))


# speedup vs baseline: 20.4069x; 1.1103x over previous
import functools

import jax
import jax.numpy as jnp
from jax import lax
from jax.experimental import pallas as pl
from jax.experimental.pallas import tpu as pltpu
from jax.experimental.pallas import tpu_sc as plsc

N = 10000
E = 160000
K = 64
CH = 160
CH2 = CH // 2
EPAD = 16 * CH * K
NACC = 10240
SLICE = NACC // 16
CO = 624
REM = N - 16 * CO

_mesh = plsc.VectorSubcoreMesh(core_axis_name="c", subcore_axis_name="s")


@functools.partial(
    pl.kernel,
    out_type=jax.ShapeDtypeStruct((NACC,), jnp.float32),
    mesh=_mesh,
    scratch_types=[
        pltpu.VMEM((CH, K), jnp.int32),
        pltpu.VMEM((K,), jnp.float32),
        pltpu.VMEM((SLICE,), jnp.float32),
        pltpu.VMEM_SHARED((NACC,), jnp.float32),
        pltpu.SemaphoreType.DMA,
    ],
)
def _deg_kernel(dst_hbm, out_hbm, dst_buf, ones_buf, res_buf, acc_sh, sem):
    c = lax.axis_index("c")
    s = lax.axis_index("s")
    for i in range(K // 16):
        ones_buf[pl.ds(i * 16, 16)] = jnp.full((16,), 1.0, jnp.float32)
    for i in range(SLICE // 16):
        res_buf[pl.ds(i * 16, 16)] = jnp.zeros((16,), jnp.float32)
    pltpu.sync_copy(res_buf, acc_sh.at[pl.ds(s * SLICE, SLICE)])
    plsc.subcore_barrier()
    pltpu.sync_copy(dst_hbm.at[s], dst_buf)

    def body(j, carry):
        pltpu.sync_copy(ones_buf, acc_sh.at[dst_buf.at[j]], add=True)
        return carry

    lax.fori_loop(0, CH, body, 0)
    plsc.subcore_barrier()
    pltpu.sync_copy(acc_sh.at[pl.ds(s * SLICE, SLICE)], res_buf)
    for i in range(SLICE // 16):
        res_buf[pl.ds(i * 16, 16)] = res_buf[pl.ds(i * 16, 16)] + 1.0

    @pl.when(c == 0)
    def _():
        pltpu.sync_copy(res_buf, out_hbm.at[pl.ds(s * SLICE, SLICE)])


def _edge_loop(g_hbm, src_row, dst_row, acc_sh, sidx, didx, rows,
               msi, mdi, mr, n):
    NS = 5

    def idx_issue(j, b):
        pltpu.async_copy(src_row(j), sidx[b], msi[b])
        pltpu.async_copy(dst_row(j), didx[b], mdi[b])

    def idx_wait(b):
        pltpu.make_async_copy(src_row(0), sidx[b], msi[b]).wait()
        pltpu.make_async_copy(dst_row(0), didx[b], mdi[b]).wait()

    def gather_issue(b):
        pltpu.async_copy(g_hbm.at[sidx[b]], rows[b], mr[b])

    for j in range(4):
        idx_issue(j, j)
    for j in range(2):
        idx_wait(j)
        gather_issue(j)

    def body(t, carry):
        j0 = t * NS
        for b in range(NS):
            j = j0 + b
            b2 = (b + 2) % NS
            b4 = (b + 4) % NS

            @pl.when(j + 4 < n)
            def _():
                idx_issue(j + 4, b4)

            @pl.when(j + 2 < n)
            def _():
                idx_wait(b2)
                gather_issue(b2)

            pltpu.make_async_copy(g_hbm.at[sidx[b]], rows[b], mr[b]).wait()
            pltpu.sync_copy(rows[b], acc_sh.at[didx[b]], add=True)
        return carry

    lax.fori_loop(0, n // NS, body, 0)


_SCAT_SCRATCH = ([pltpu.VMEM((K,), jnp.int32)] * 5
                 + [pltpu.VMEM((K,), jnp.int32)] * 5
                 + [pltpu.VMEM((K, 128), jnp.float32)] * 5
                 + [pltpu.VMEM_SHARED((NACC, 128), jnp.float32)]
                 + [pltpu.SemaphoreType.DMA] * 15)


@functools.partial(
    pl.kernel,
    out_type=jax.ShapeDtypeStruct((2 * N, 128), jnp.float32),
    mesh=_mesh,
    scratch_types=_SCAT_SCRATCH,
)
def _scatter_fsplit(g_hbm, src_hbm, dst_hbm, out_hbm,
                    s0, s1, s2, s3, s4, d0, d1, d2, d3, d4,
                    r0, r1, r2, r3, r4, acc_sh,
                    a0, a1, a2, a3, a4, e0, e1, e2, e3, e4,
                    f0, f1, f2, f3, f4):
    c = lax.axis_index("c")
    s = lax.axis_index("s")
    base = c * N
    pltpu.sync_copy(g_hbm.at[pl.ds(base + s * CO, CO)],
                    acc_sh.at[pl.ds(s * CO, CO)])

    @pl.when(s == 0)
    def _():
        pltpu.sync_copy(g_hbm.at[pl.ds(base + 16 * CO, REM)],
                        acc_sh.at[pl.ds(16 * CO, REM)])

    plsc.subcore_barrier()
    _edge_loop(g_hbm,
               lambda j: src_hbm.at[c, s, j],
               lambda j: dst_hbm.at[s, j],
               acc_sh, (s0, s1, s2, s3, s4), (d0, d1, d2, d3, d4),
               (r0, r1, r2, r3, r4), (a0, a1, a2, a3, a4),
               (e0, e1, e2, e3, e4), (f0, f1, f2, f3, f4), CH)
    plsc.subcore_barrier()
    pltpu.sync_copy(acc_sh.at[pl.ds(s * CO, CO)],
                    out_hbm.at[pl.ds(base + s * CO, CO)])

    @pl.when(s == 0)
    def _():
        pltpu.sync_copy(acc_sh.at[pl.ds(16 * CO, REM)],
                        out_hbm.at[pl.ds(base + 16 * CO, REM)])


@functools.partial(
    pl.kernel,
    out_type=jax.ShapeDtypeStruct((2 * N, 128), jnp.float32),
    mesh=_mesh,
    scratch_types=_SCAT_SCRATCH,
)
def _scatter_esplit(g_hbm, src_hbm, dst_hbm, out_hbm,
                    s0, s1, s2, s3, s4, d0, d1, d2, d3, d4,
                    r0, r1, r2, r3, r4, acc_sh,
                    a0, a1, a2, a3, a4, e0, e1, e2, e3, e4,
                    f0, f1, f2, f3, f4):
    c = lax.axis_index("c")
    s = lax.axis_index("s")
    base = c * N
    for r in range(K):
        for q in range(8):
            r0[r, pl.ds(q * 16, 16)] = jnp.zeros((16,), jnp.float32)
    for t in range(SLICE // K):
        pltpu.sync_copy(r0, acc_sh.at[pl.ds(s * SLICE + t * K, K)])
    plsc.subcore_barrier()
    _edge_loop(g_hbm,
               lambda j: src_hbm.at[c, s, j],
               lambda j: dst_hbm.at[c, s, j],
               acc_sh, (s0, s1, s2, s3, s4), (d0, d1, d2, d3, d4),
               (r0, r1, r2, r3, r4), (a0, a1, a2, a3, a4),
               (e0, e1, e2, e3, e4), (f0, f1, f2, f3, f4), CH2)
    plsc.subcore_barrier()
    pltpu.sync_copy(acc_sh.at[pl.ds(s * CO, CO)],
                    out_hbm.at[pl.ds(base + s * CO, CO)])

    @pl.when(s == 0)
    def _():
        pltpu.sync_copy(acc_sh.at[pl.ds(16 * CO, REM)],
                        out_hbm.at[pl.ds(base + 16 * CO, REM)])


_RB = 1000


def _tc_first(x, deg, w1):
    f_in, f_out = w1.shape
    dh = f_out // 2

    def body(x_ref, deg_ref, w_ref, g_ref, dinv_ref):
        dv = lax.rsqrt(deg_ref[...])
        m = jnp.dot(x_ref[...], w_ref[...], preferred_element_type=jnp.float32)
        g_ref[...] = (m * dv)[None]
        dinv_ref[...] = dv

    return pl.pallas_call(
        body,
        grid=(N // _RB, 2),
        in_specs=[
            pl.BlockSpec((_RB, f_in), lambda r, c: (r, 0)),
            pl.BlockSpec((_RB, 1), lambda r, c: (r, 0)),
            pl.BlockSpec((f_in, dh), lambda r, c: (0, c)),
        ],
        out_specs=[
            pl.BlockSpec((1, _RB, dh), lambda r, c: (c, r, 0)),
            pl.BlockSpec((_RB, 1), lambda r, c: (r, 0)),
        ],
        out_shape=[
            jax.ShapeDtypeStruct((2, N, dh), jnp.float32),
            jax.ShapeDtypeStruct((N, 1), jnp.float32),
        ],
    )(x, deg, w1)


def _tc_second(s1, dinv, b, w):
    f_in, f_out = w.shape

    def body(sa_ref, sb_ref, dinv_ref, b_ref, w_ref, g_ref):
        dv = dinv_ref[...]
        h = jnp.concatenate([sa_ref[...], sb_ref[...]], axis=1) * dv + b_ref[...]
        h = jnp.maximum(h, 0.0)
        g_ref[...] = jnp.dot(h, w_ref[...],
                             preferred_element_type=jnp.float32) * dv

    return pl.pallas_call(
        body,
        grid=(N // _RB,),
        in_specs=[
            pl.BlockSpec((_RB, 128), lambda r: (r, 0)),
            pl.BlockSpec((_RB, 128), lambda r: (N // _RB + r, 0)),
            pl.BlockSpec((_RB, 1), lambda r: (r, 0)),
            pl.BlockSpec((1, f_in), lambda r: (0, 0)),
            pl.BlockSpec((f_in, f_out), lambda r: (0, 0)),
        ],
        out_specs=pl.BlockSpec((_RB, f_out), lambda r: (r, 0)),
        out_shape=jax.ShapeDtypeStruct((N, f_out), jnp.float32),
    )(s1, s1, dinv, b, w)


def _tc_third(p2, g, dinv, b):
    f = g.shape[1]

    def body(pa_ref, pb_ref, g_ref, dinv_ref, b_ref, o_ref):
        dv = dinv_ref[...]
        s = pa_ref[...] + pb_ref[...] + g_ref[...]
        h = jnp.maximum(s * dv + b_ref[...], 0.0)
        o_ref[...] = h * dv

    return pl.pallas_call(
        body,
        grid=(N // _RB,),
        in_specs=[
            pl.BlockSpec((_RB, f), lambda r: (r, 0)),
            pl.BlockSpec((_RB, f), lambda r: (N // _RB + r, 0)),
            pl.BlockSpec((_RB, f), lambda r: (r, 0)),
            pl.BlockSpec((_RB, 1), lambda r: (r, 0)),
            pl.BlockSpec((1, f), lambda r: (0, 0)),
        ],
        out_specs=pl.BlockSpec((_RB, f), lambda r: (r, 0)),
        out_shape=jax.ShapeDtypeStruct((N, f), jnp.float32),
    )(p2, p2, g, dinv, b)


def _tc_last(p3, g, dinv, w, b):
    f_in, f_out = w.shape

    def body(pa_ref, pb_ref, g_ref, dinv_ref, w_ref, b_ref, o_ref):
        s = pa_ref[...] + pb_ref[...] + g_ref[...]
        m = jnp.dot(s, w_ref[...], preferred_element_type=jnp.float32)
        o_ref[...] = m * dinv_ref[...] + b_ref[...]

    return pl.pallas_call(
        body,
        grid=(N // _RB,),
        in_specs=[
            pl.BlockSpec((_RB, f_in), lambda r: (r, 0)),
            pl.BlockSpec((_RB, f_in), lambda r: (N // _RB + r, 0)),
            pl.BlockSpec((_RB, f_in), lambda r: (r, 0)),
            pl.BlockSpec((_RB, 1), lambda r: (r, 0)),
            pl.BlockSpec((f_in, f_out), lambda r: (0, 0)),
            pl.BlockSpec((1, f_out), lambda r: (0, 0)),
        ],
        out_specs=pl.BlockSpec((_RB, f_out), lambda r: (r, 0)),
        out_shape=jax.ShapeDtypeStruct((N, f_out), jnp.float32),
    )(p3, p3, g, dinv, w, b)


def kernel(x, edge_index, W1, b1, W2, b2, W3, b3):
    src = edge_index[0].astype(jnp.int32)
    dst = edge_index[1].astype(jnp.int32)
    pad = EPAD - E
    pad_src = (jnp.arange(pad, dtype=jnp.int32) * 37) % N
    pad_dst = N + (jnp.arange(pad, dtype=jnp.int32) % (NACC - N))
    src_p = jnp.concatenate([src, pad_src])
    dst_p = jnp.concatenate([dst, pad_dst])
    src2 = jnp.stack([src_p, src_p + N]).reshape(2, 16, CH, K)
    dst3 = dst_p.reshape(16, CH, K)
    src_h = src_p.reshape(2, 16, CH2, K)
    dst_h = dst_p.reshape(2, 16, CH2, K)

    deg = _deg_kernel(dst3)[:N].reshape(N, 1)

    g1, dinv = _tc_first(x, deg, W1)
    s1 = _scatter_fsplit(g1.reshape(2 * N, 128), src2, dst3)
    g2 = _tc_second(s1, dinv, b1.reshape(1, -1), W2)
    p2 = _scatter_esplit(g2, src_h, dst_h)
    g3 = _tc_third(p2, g2, dinv, b2.reshape(1, -1))
    p3 = _scatter_esplit(g3, src_h, dst_h)
    return _tc_last(p3, g3, dinv, W3, b3.reshape(1, -1))


# speedup vs baseline: 21.1652x; 1.0372x over previous
import functools

import jax
import jax.numpy as jnp
from jax import lax
from jax.experimental import pallas as pl
from jax.experimental.pallas import tpu as pltpu
from jax.experimental.pallas import tpu_sc as plsc

N = 10000
E = 160000
K = 64
CH = 160
CH2 = CH // 2
EPAD = 16 * CH * K
NACC = 10240
SLICE = NACC // 16
CO = 624
REM = N - 16 * CO

_mesh = plsc.VectorSubcoreMesh(core_axis_name="c", subcore_axis_name="s")


@functools.partial(
    pl.kernel,
    out_type=jax.ShapeDtypeStruct((NACC,), jnp.float32),
    mesh=_mesh,
    scratch_types=[
        pltpu.VMEM((CH, K), jnp.int32),
        pltpu.VMEM((K,), jnp.float32),
        pltpu.VMEM((SLICE,), jnp.float32),
        pltpu.VMEM_SHARED((NACC,), jnp.float32),
        pltpu.SemaphoreType.DMA,
    ],
)
def _deg_kernel(dst_hbm, out_hbm, dst_buf, ones_buf, res_buf, acc_sh, sem):
    c = lax.axis_index("c")
    s = lax.axis_index("s")
    for i in range(K // 16):
        ones_buf[pl.ds(i * 16, 16)] = jnp.full((16,), 1.0, jnp.float32)
    for i in range(SLICE // 16):
        res_buf[pl.ds(i * 16, 16)] = jnp.zeros((16,), jnp.float32)
    pltpu.sync_copy(res_buf, acc_sh.at[pl.ds(s * SLICE, SLICE)])
    plsc.subcore_barrier()
    pltpu.sync_copy(dst_hbm.at[s], dst_buf)

    def body(j, carry):
        pltpu.sync_copy(ones_buf, acc_sh.at[dst_buf.at[j]], add=True)
        return carry

    lax.fori_loop(0, CH, body, 0)
    plsc.subcore_barrier()
    pltpu.sync_copy(acc_sh.at[pl.ds(s * SLICE, SLICE)], res_buf)
    for i in range(SLICE // 16):
        res_buf[pl.ds(i * 16, 16)] = res_buf[pl.ds(i * 16, 16)] + 1.0

    @pl.when(c == 0)
    def _():
        pltpu.sync_copy(res_buf, out_hbm.at[pl.ds(s * SLICE, SLICE)])


def _edge_loop(g_hbm, src_row, dst_row, fix_idx, acc_sh, sidx, didx, rows,
               msi, mdi, mr, n):
    NS = 5

    def idx_issue(j, b):
        pltpu.async_copy(src_row(j), sidx[b], msi[b])
        pltpu.async_copy(dst_row(j), didx[b], mdi[b])

    def idx_wait(b):
        pltpu.make_async_copy(src_row(0), sidx[b], msi[b]).wait()
        pltpu.make_async_copy(dst_row(0), didx[b], mdi[b]).wait()

    def gather_issue(b):
        pltpu.async_copy(g_hbm.at[sidx[b]], rows[b], mr[b])

    for j in range(4):
        idx_issue(j, j)
    for j in range(2):
        idx_wait(j)
        fix_idx(j)
        gather_issue(j)

    def body(t, carry):
        j0 = t * NS
        for b in range(NS):
            j = j0 + b
            b2 = (b + 2) % NS
            b4 = (b + 4) % NS

            @pl.when(j + 4 < n)
            def _():
                idx_issue(j + 4, b4)

            @pl.when(j + 2 < n)
            def _():
                idx_wait(b2)
                fix_idx(b2)
                gather_issue(b2)

            pltpu.make_async_copy(g_hbm.at[sidx[b]], rows[b], mr[b]).wait()
            pltpu.sync_copy(rows[b], acc_sh.at[didx[b]], add=True)
        return carry

    lax.fori_loop(0, n // NS, body, 0)


_SCAT_SCRATCH = ([pltpu.VMEM((K,), jnp.int32)] * 5
                 + [pltpu.VMEM((K,), jnp.int32)] * 5
                 + [pltpu.VMEM((K, 128), jnp.float32)] * 5
                 + [pltpu.VMEM_SHARED((NACC, 128), jnp.float32)]
                 + [pltpu.SemaphoreType.DMA] * 15)


@functools.partial(
    pl.kernel,
    out_type=jax.ShapeDtypeStruct((2 * N, 128), jnp.float32),
    mesh=_mesh,
    scratch_types=_SCAT_SCRATCH,
)
def _scatter_fsplit(g_hbm, src_hbm, dst_hbm, out_hbm,
                    s0, s1, s2, s3, s4, d0, d1, d2, d3, d4,
                    r0, r1, r2, r3, r4, acc_sh,
                    a0, a1, a2, a3, a4, e0, e1, e2, e3, e4,
                    f0, f1, f2, f3, f4):
    c = lax.axis_index("c")
    s = lax.axis_index("s")
    base = c * N
    sidx = (s0, s1, s2, s3, s4)
    pltpu.sync_copy(g_hbm.at[pl.ds(base + s * CO, CO)],
                    acc_sh.at[pl.ds(s * CO, CO)])

    @pl.when(s == 0)
    def _():
        pltpu.sync_copy(g_hbm.at[pl.ds(base + 16 * CO, REM)],
                        acc_sh.at[pl.ds(16 * CO, REM)])

    plsc.subcore_barrier()

    def fix_idx(b):
        sb = sidx[b]
        for q in range(K // 16):
            sb[pl.ds(q * 16, 16)] = sb[pl.ds(q * 16, 16)] + base

    _edge_loop(g_hbm,
               lambda j: src_hbm.at[s, j],
               lambda j: dst_hbm.at[s, j],
               fix_idx,
               acc_sh, sidx, (d0, d1, d2, d3, d4),
               (r0, r1, r2, r3, r4), (a0, a1, a2, a3, a4),
               (e0, e1, e2, e3, e4), (f0, f1, f2, f3, f4), CH)
    plsc.subcore_barrier()
    pltpu.sync_copy(acc_sh.at[pl.ds(s * CO, CO)],
                    out_hbm.at[pl.ds(base + s * CO, CO)])

    @pl.when(s == 0)
    def _():
        pltpu.sync_copy(acc_sh.at[pl.ds(16 * CO, REM)],
                        out_hbm.at[pl.ds(base + 16 * CO, REM)])


@functools.partial(
    pl.kernel,
    out_type=jax.ShapeDtypeStruct((2 * N, 128), jnp.float32),
    mesh=_mesh,
    scratch_types=_SCAT_SCRATCH,
)
def _scatter_esplit(g_hbm, src_hbm, dst_hbm, out_hbm,
                    s0, s1, s2, s3, s4, d0, d1, d2, d3, d4,
                    r0, r1, r2, r3, r4, acc_sh,
                    a0, a1, a2, a3, a4, e0, e1, e2, e3, e4,
                    f0, f1, f2, f3, f4):
    c = lax.axis_index("c")
    s = lax.axis_index("s")
    base = c * N
    j0 = c * CH2
    for r in range(K):
        for q in range(8):
            r0[r, pl.ds(q * 16, 16)] = jnp.zeros((16,), jnp.float32)
    for t in range(SLICE // K):
        pltpu.sync_copy(r0, acc_sh.at[pl.ds(s * SLICE + t * K, K)])
    plsc.subcore_barrier()
    _edge_loop(g_hbm,
               lambda j: src_hbm.at[s, j0 + j],
               lambda j: dst_hbm.at[s, j0 + j],
               lambda b: None,
               acc_sh, (s0, s1, s2, s3, s4), (d0, d1, d2, d3, d4),
               (r0, r1, r2, r3, r4), (a0, a1, a2, a3, a4),
               (e0, e1, e2, e3, e4), (f0, f1, f2, f3, f4), CH2)
    plsc.subcore_barrier()
    pltpu.sync_copy(acc_sh.at[pl.ds(s * CO, CO)],
                    out_hbm.at[pl.ds(base + s * CO, CO)])

    @pl.when(s == 0)
    def _():
        pltpu.sync_copy(acc_sh.at[pl.ds(16 * CO, REM)],
                        out_hbm.at[pl.ds(base + 16 * CO, REM)])


_RB = 1000
_RB2 = 2000


def _tc_first(x, deg, w1):
    f_in, f_out = w1.shape
    dh = f_out // 2

    def body(x_ref, deg_ref, w_ref, g_ref):
        m = jnp.dot(x_ref[...], w_ref[...], preferred_element_type=jnp.float32)
        g_ref[...] = m * lax.rsqrt(deg_ref[...])

    return pl.pallas_call(
        body,
        grid=(N // _RB, 2),
        in_specs=[
            pl.BlockSpec((_RB, f_in), lambda r, c: (r, 0)),
            pl.BlockSpec((_RB, 1), lambda r, c: (r, 0)),
            pl.BlockSpec((f_in, dh), lambda r, c: (0, c)),
        ],
        out_specs=pl.BlockSpec((_RB, dh), lambda r, c: (c * (N // _RB) + r, 0)),
        out_shape=jax.ShapeDtypeStruct((2 * N, dh), jnp.float32),
    )(x, deg, w1)


def _tc_second(s1, deg, b, w):
    f_in, f_out = w.shape

    def body(sa_ref, sb_ref, deg_ref, b_ref, w_ref, g_ref):
        dv = lax.rsqrt(deg_ref[...])
        h = jnp.concatenate([sa_ref[...], sb_ref[...]], axis=1) * dv + b_ref[...]
        h = jnp.maximum(h, 0.0)
        g_ref[...] = jnp.dot(h, w_ref[...],
                             preferred_element_type=jnp.float32) * dv

    return pl.pallas_call(
        body,
        grid=(N // _RB2,),
        in_specs=[
            pl.BlockSpec((_RB2, 128), lambda r: (r, 0)),
            pl.BlockSpec((_RB2, 128), lambda r: (N // _RB2 + r, 0)),
            pl.BlockSpec((_RB2, 1), lambda r: (r, 0)),
            pl.BlockSpec((1, f_in), lambda r: (0, 0)),
            pl.BlockSpec((f_in, f_out), lambda r: (0, 0)),
        ],
        out_specs=pl.BlockSpec((_RB2, f_out), lambda r: (r, 0)),
        out_shape=jax.ShapeDtypeStruct((N, f_out), jnp.float32),
    )(s1, s1, deg, b, w)


def _tc_third(p2, g, deg, b):
    f = g.shape[1]

    def body(pa_ref, pb_ref, g_ref, deg_ref, b_ref, o_ref):
        dv = lax.rsqrt(deg_ref[...])
        s = pa_ref[...] + pb_ref[...] + g_ref[...]
        h = jnp.maximum(s * dv + b_ref[...], 0.0)
        o_ref[...] = h * dv

    return pl.pallas_call(
        body,
        grid=(N // _RB2,),
        in_specs=[
            pl.BlockSpec((_RB2, f), lambda r: (r, 0)),
            pl.BlockSpec((_RB2, f), lambda r: (N // _RB2 + r, 0)),
            pl.BlockSpec((_RB2, f), lambda r: (r, 0)),
            pl.BlockSpec((_RB2, 1), lambda r: (r, 0)),
            pl.BlockSpec((1, f), lambda r: (0, 0)),
        ],
        out_specs=pl.BlockSpec((_RB2, f), lambda r: (r, 0)),
        out_shape=jax.ShapeDtypeStruct((N, f), jnp.float32),
    )(p2, p2, g, deg, b)


def _tc_last(p3, g, deg, w, b):
    f_in, f_out = w.shape

    def body(pa_ref, pb_ref, g_ref, deg_ref, w_ref, b_ref, o_ref):
        s = pa_ref[...] + pb_ref[...] + g_ref[...]
        m = jnp.dot(s, w_ref[...], preferred_element_type=jnp.float32)
        o_ref[...] = m * lax.rsqrt(deg_ref[...]) + b_ref[...]

    return pl.pallas_call(
        body,
        grid=(N // _RB2,),
        in_specs=[
            pl.BlockSpec((_RB2, f_in), lambda r: (r, 0)),
            pl.BlockSpec((_RB2, f_in), lambda r: (N // _RB2 + r, 0)),
            pl.BlockSpec((_RB2, f_in), lambda r: (r, 0)),
            pl.BlockSpec((_RB2, 1), lambda r: (r, 0)),
            pl.BlockSpec((f_in, f_out), lambda r: (0, 0)),
            pl.BlockSpec((1, f_out), lambda r: (0, 0)),
        ],
        out_specs=pl.BlockSpec((_RB2, f_out), lambda r: (r, 0)),
        out_shape=jax.ShapeDtypeStruct((N, f_out), jnp.float32),
    )(p3, p3, g, deg, w, b)


def kernel(x, edge_index, W1, b1, W2, b2, W3, b3):
    src = edge_index[0].astype(jnp.int32)
    dst = edge_index[1].astype(jnp.int32)
    pad = EPAD - E
    pad_src = (jnp.arange(pad, dtype=jnp.int32) * 37) % N
    pad_dst = N + (jnp.arange(pad, dtype=jnp.int32) % (NACC - N))
    src3 = jnp.concatenate([src, pad_src]).reshape(16, CH, K)
    dst3 = jnp.concatenate([dst, pad_dst]).reshape(16, CH, K)

    deg = _deg_kernel(dst3).reshape(NACC, 1)

    g1 = _tc_first(x, deg, W1)
    s1 = _scatter_fsplit(g1, src3, dst3)
    g2 = _tc_second(s1, deg, b1.reshape(1, -1), W2)
    p2 = _scatter_esplit(g2, src3, dst3)
    g3 = _tc_third(p2, g2, deg, b2.reshape(1, -1))
    p3 = _scatter_esplit(g3, src3, dst3)
    return _tc_last(p3, g3, deg, W3, b3.reshape(1, -1))
